# baseline (speedup 1.0000x reference)
import functools

import jax
import jax.numpy as jnp
from jax import lax
from jax.experimental import pallas as pl
from jax.experimental.pallas import tpu as pltpu
from jax.experimental.pallas import tpu_sc as plsc

D_MODEL = 1024
HIDDEN = 4096
NUM_EXPERTS = 8
N_TOKENS = 4096

ROW_BLK = 256
P_ROWS = 2 * N_TOKENS + NUM_EXPERTS * ROW_BLK
NB = P_ROWS // ROW_BLK
HID_BLK = 4096

NW = 32
TPW = N_TOKENS // NW
CSUM_BLK = 512


def _router_kernel(x_ref, wr_ref, pos_ref, w2_ref, gexp_ref, lbal_ref):
    x = x_ref[...]
    wr = wr_ref[...]
    logits = jnp.dot(x, wr, preferred_element_type=jnp.float32)
    lane = lax.broadcasted_iota(jnp.int32, logits.shape, 1)

    m1 = jnp.max(logits, axis=-1, keepdims=True)
    e1 = jnp.min(jnp.where(logits == m1, lane, NUM_EXPERTS), axis=-1,
                 keepdims=True)
    oh1 = (lane == e1)
    masked = jnp.where(oh1, -jnp.inf, logits)
    m2 = jnp.max(masked, axis=-1, keepdims=True)
    e2 = jnp.min(jnp.where(masked == m2, lane, NUM_EXPERTS), axis=-1,
                 keepdims=True)
    oh2 = (lane == e2)
    oh1f = oh1.astype(jnp.float32)
    oh2f = oh2.astype(jnp.float32)

    a = jnp.exp(m2 - m1)
    wa = 1.0 / (1.0 + a)
    wb = a / (1.0 + a)
    w2_ref[...] = jnp.concatenate([wa, wb], axis=-1)

    z = jnp.exp(logits - m1)
    probs = z / jnp.sum(z, axis=-1, keepdims=True)
    rppe = jnp.mean(probs, axis=0)
    tpe = jnp.mean(oh1f + oh2f, axis=0)
    lbal_ref[0, 0] = NUM_EXPERTS * jnp.sum(tpe * rppe)

    h = oh1f + oh2f
    r_i = lax.broadcasted_iota(jnp.int32, (CSUM_BLK, CSUM_BLK), 0)
    c_i = lax.broadcasted_iota(jnp.int32, (CSUM_BLK, CSUM_BLK), 1)
    tri = (c_i < r_i).astype(jnp.float32)
    carry = jnp.zeros((1, NUM_EXPERTS), jnp.float32)
    excl_chunks = []
    for q in range(N_TOKENS // CSUM_BLK):
        hq = lax.slice_in_dim(h, q * CSUM_BLK, (q + 1) * CSUM_BLK, axis=0)
        excl_chunks.append(
            jnp.dot(tri, hq, preferred_element_type=jnp.float32) + carry)
        carry = carry + jnp.sum(hq, axis=0, keepdims=True)
    excl = jnp.concatenate(excl_chunks, axis=0)
    counts = carry

    cnt_i = counts.astype(jnp.int32)
    cnt_pad = ((cnt_i + (ROW_BLK - 1)) // ROW_BLK) * ROW_BLK
    cnt_pad_f = cnt_pad.astype(jnp.float32)
    r8 = lax.broadcasted_iota(jnp.int32, (NUM_EXPERTS, NUM_EXPERTS), 0)
    c8 = lax.broadcasted_iota(jnp.int32, (NUM_EXPERTS, NUM_EXPERTS), 1)
    strict8 = (r8 < c8).astype(jnp.float32)
    base = jnp.dot(cnt_pad_f, strict8,
                   preferred_element_type=jnp.float32)
    ends = base + cnt_pad_f

    base_b = jnp.broadcast_to(base, excl.shape)
    rank1 = jnp.sum(jnp.where(oh1, excl + base_b, 0.0), axis=-1, keepdims=True)
    rank2 = jnp.sum(jnp.where(oh2, excl + base_b, 0.0), axis=-1, keepdims=True)
    pos_ref[...] = jnp.concatenate([rank1, rank2], axis=-1).astype(jnp.int32)

    blk_start = (lax.broadcasted_iota(jnp.int32, (1, NB), 1)
                 * ROW_BLK).astype(jnp.float32)
    acc = jnp.zeros((1, NB), jnp.int32)
    lane8 = lax.broadcasted_iota(jnp.int32, (1, NUM_EXPERTS), 1)
    for e in range(NUM_EXPERTS):
        end_e = jnp.sum(jnp.where(lane8 == e, ends, 0.0))
        acc = acc + (blk_start >= end_e).astype(jnp.int32)
    gexp_ref[...] = jnp.minimum(acc, NUM_EXPERTS - 1)


def _route(x_flat, Wr):
    return pl.pallas_call(
        _router_kernel,
        out_shape=(
            jax.ShapeDtypeStruct((N_TOKENS, 2), jnp.int32),
            jax.ShapeDtypeStruct((N_TOKENS, 2), jnp.float32),
            jax.ShapeDtypeStruct((1, NB), jnp.int32),
            jax.ShapeDtypeStruct((1, 1), jnp.float32),
        ),
        in_specs=[
            pl.BlockSpec(memory_space=pltpu.VMEM),
            pl.BlockSpec(memory_space=pltpu.VMEM),
        ],
        out_specs=(
            pl.BlockSpec(memory_space=pltpu.VMEM),
            pl.BlockSpec(memory_space=pltpu.VMEM),
            pl.BlockSpec(memory_space=pltpu.VMEM),
            pl.BlockSpec(memory_space=pltpu.SMEM),
        ),
    )(x_flat, Wr)


def _dispatch_body(x_hbm, pos_hbm, w2_hbm, xg_hbm, wg_hbm, idxv, wv, xbuf,
                   sem):
    w = lax.axis_index("s") * 2 + lax.axis_index("c")
    pltpu.sync_copy(pos_hbm.at[0, w], idxv.at[0])
    pltpu.sync_copy(pos_hbm.at[1, w], idxv.at[1])
    pltpu.sync_copy(w2_hbm.at[0, w], wv.at[0])
    pltpu.sync_copy(w2_hbm.at[1, w], wv.at[1])
    for c in range(4):
        pltpu.sync_copy(x_hbm.at[pl.ds(w * TPW + c * 32, 32)], xbuf)
        cp0 = pltpu.async_copy(xbuf, xg_hbm.at[idxv.at[0, c]], sem)
        cp1 = pltpu.async_copy(xbuf, xg_hbm.at[idxv.at[1, c]], sem)
        cp2 = pltpu.async_copy(wv.at[0, c], wg_hbm.at[idxv.at[0, c]], sem)
        cp3 = pltpu.async_copy(wv.at[1, c], wg_hbm.at[idxv.at[1, c]], sem)
        cp0.wait()
        cp1.wait()
        cp2.wait()
        cp3.wait()


def _dispatch(x_flat, posB, w2B):
    mesh = plsc.VectorSubcoreMesh(core_axis_name="c", subcore_axis_name="s",
                                  num_cores=2, num_subcores=16)
    return pl.kernel(
        _dispatch_body,
        out_type=(
            jax.ShapeDtypeStruct((P_ROWS, D_MODEL // 2), jnp.int32),
            jax.ShapeDtypeStruct((P_ROWS,), jnp.float32),
        ),
        mesh=mesh,
        scratch_types=[
            pltpu.VMEM((2, 4, 32), jnp.int32),
            pltpu.VMEM((2, 4, 32), jnp.float32),
            pltpu.VMEM((32, D_MODEL // 2), jnp.int32),
            pltpu.SemaphoreType.DMA,
        ],
    )(x_flat, posB, w2B)


def _gmm_kernel(g_ref, xg_ref, w1_ref, b1_ref, wg_ref, y_ref, w1bf_ref):
    i = pl.program_id(1)
    g_now = g_ref[i]
    g_prev = g_ref[jnp.maximum(i - 1, 0)]

    @pl.when((i == 0) | (g_now != g_prev))
    def _():
        w1bf_ref[...] = w1_ref[0].astype(jnp.bfloat16)

    xi = xg_ref[...]
    lo = lax.bitcast_convert_type(xi << 16, jnp.float32)
    hi = lax.bitcast_convert_type(xi & jnp.int32(-65536), jnp.float32)
    xbf = jnp.concatenate([lo, hi], axis=1).astype(jnp.bfloat16)
    y = jnp.dot(xbf, w1bf_ref[...], preferred_element_type=jnp.float32)
    y_ref[...] = (y + b1_ref[0]) * wg_ref[...]


def _grouped_matmul(gexp_flat, xg, W1, b1, wg):
    grid_spec = pltpu.PrefetchScalarGridSpec(
        num_scalar_prefetch=1,
        grid=(HIDDEN // HID_BLK, NB),
        in_specs=[
            pl.BlockSpec((ROW_BLK, D_MODEL // 2), lambda j, i, g: (i, 0)),
            pl.BlockSpec((1, D_MODEL, HID_BLK), lambda j, i, g: (g[i], 0, j)),
            pl.BlockSpec((1, 1, HID_BLK), lambda j, i, g: (g[i], 0, j)),
            pl.BlockSpec((ROW_BLK, 1), lambda j, i, g: (i, 0)),
        ],
        out_specs=pl.BlockSpec((ROW_BLK, HID_BLK), lambda j, i, g: (i, j)),
        scratch_shapes=[pltpu.VMEM((D_MODEL, HID_BLK), jnp.bfloat16)],
    )
    return pl.pallas_call(
        _gmm_kernel,
        grid_spec=grid_spec,
        out_shape=jax.ShapeDtypeStruct((P_ROWS, HIDDEN), jnp.float32),
        compiler_params=pltpu.CompilerParams(
            dimension_semantics=("arbitrary", "arbitrary"),
        ),
    )(gexp_flat, xg, W1, b1.reshape(NUM_EXPERTS, 1, HIDDEN),
      wg.reshape(P_ROWS, 1))


NCH = TPW // 4


def _combine_body(y_hbm, pos_hbm, out_hbm, idxv, rA0, rA1, rB0, rB1, o0, o1,
                  sA0, sA1, sB0, sB1, sO0, sO1):
    w = lax.axis_index("s") * 2 + lax.axis_index("c")
    pltpu.sync_copy(pos_hbm.at[0, w], idxv.at[0])
    pltpu.sync_copy(pos_hbm.at[1, w], idxv.at[1])
    rA = (rA0, rA1)
    rB = (rB0, rB1)
    ob = (o0, o1)
    sA = (sA0, sA1)
    sB = (sB0, sB1)
    sO = (sO0, sO1)

    def fire(ch, b):
        pltpu.async_copy(y_hbm.at[idxv.at[0, pl.ds(ch * 4, 4)]], rA[b], sA[b])
        pltpu.async_copy(y_hbm.at[idxv.at[1, pl.ds(ch * 4, 4)]], rB[b], sB[b])

    fire(0, 0)
    fire(1, 1)

    def outer(i, _):
        for b in range(2):
            ch = i * 2 + b
            pltpu.make_async_copy(
                y_hbm.at[idxv.at[0, pl.ds(ch * 4, 4)]], rA[b], sA[b]).wait()
            pltpu.make_async_copy(
                y_hbm.at[idxv.at[1, pl.ds(ch * 4, 4)]], rB[b], sB[b]).wait()

            @pl.when(i > 0)
            def _():
                pltpu.make_async_copy(
                    ob[b], out_hbm.at[pl.ds(w * TPW + (ch - 2) * 4, 4)],
                    sO[b]).wait()

            for t in range(4):

                def elem_body(j, _):
                    for q in range(4):
                        sl = pl.ds(j * 64 + q * 16, 16)
                        ob[b][t, sl] = rA[b][t, sl] + rB[b][t, sl]
                    return 0

                lax.fori_loop(0, HIDDEN // 64, elem_body, 0)

            @pl.when(ch + 2 < NCH)
            def _():
                fire(ch + 2, b)

            pltpu.async_copy(
                ob[b], out_hbm.at[pl.ds(w * TPW + ch * 4, 4)], sO[b])
        return 0

    lax.fori_loop(0, NCH // 2, outer, 0)
    for b in range(2):
        pltpu.make_async_copy(
            ob[b], out_hbm.at[pl.ds(w * TPW + (NCH - 2 + b) * 4, 4)],
            sO[b]).wait()


def _combine(y, posD):
    mesh = plsc.VectorSubcoreMesh(core_axis_name="c", subcore_axis_name="s",
                                  num_cores=2, num_subcores=16)
    return pl.kernel(
        _combine_body,
        out_type=jax.ShapeDtypeStruct((N_TOKENS, HIDDEN), jnp.float32),
        mesh=mesh,
        scratch_types=[
            pltpu.VMEM((2, TPW), jnp.int32),
            pltpu.VMEM((4, HIDDEN), jnp.float32),
            pltpu.VMEM((4, HIDDEN), jnp.float32),
            pltpu.VMEM((4, HIDDEN), jnp.float32),
            pltpu.VMEM((4, HIDDEN), jnp.float32),
            pltpu.VMEM((4, HIDDEN), jnp.float32),
            pltpu.VMEM((4, HIDDEN), jnp.float32),
            pltpu.SemaphoreType.DMA,
            pltpu.SemaphoreType.DMA,
            pltpu.SemaphoreType.DMA,
            pltpu.SemaphoreType.DMA,
            pltpu.SemaphoreType.DMA,
            pltpu.SemaphoreType.DMA,
        ],
    )(y, posD)


def kernel(x, Wr, W1, b1):
    bsz, seq, d = x.shape
    x_flat = x.reshape(N_TOKENS, d)

    pos, w2, gexp, lbal = _route(x_flat, Wr)
    posT = pos.T
    posB = posT.reshape(2, NW, 4, 32)
    posD = posT.reshape(2, NW, TPW)
    w2B = w2.T.reshape(2, NW, 4, 32)

    xbf = x_flat.astype(jnp.bfloat16)
    xi = lax.bitcast_convert_type(
        jnp.stack([xbf[:, :D_MODEL // 2], xbf[:, D_MODEL // 2:]], axis=-1),
        jnp.int32)
    xg, wg = _dispatch(xi, posB, w2B)
    y = _grouped_matmul(gexp.reshape(NB), xg, W1, b1, wg)
    out = _combine(y, posD)
    return out.reshape(bsz, seq, HIDDEN), lbal.reshape(())


# speedup vs baseline: 1.1473x; 1.1473x over previous
import functools

import jax
import jax.numpy as jnp
from jax import lax
from jax.experimental import pallas as pl
from jax.experimental.pallas import tpu as pltpu
from jax.experimental.pallas import tpu_sc as plsc

D_MODEL = 1024
HIDDEN = 4096
NUM_EXPERTS = 8
N_TOKENS = 4096

ROW_BLK = 256
P_ROWS = 2 * N_TOKENS + NUM_EXPERTS * ROW_BLK
NB = P_ROWS // ROW_BLK
HID_BLK = 4096

NW = 32
TPW = N_TOKENS // NW
CSUM_BLK = 512


def _router_kernel(x_ref, wr_ref, pos_ref, w2_ref, gexp_ref, lbal_ref):
    x = x_ref[...]
    wr = wr_ref[...]
    logits = jnp.dot(x, wr, preferred_element_type=jnp.float32)
    lane = lax.broadcasted_iota(jnp.int32, logits.shape, 1)

    m1 = jnp.max(logits, axis=-1, keepdims=True)
    e1 = jnp.min(jnp.where(logits == m1, lane, NUM_EXPERTS), axis=-1,
                 keepdims=True)
    oh1 = (lane == e1)
    masked = jnp.where(oh1, -jnp.inf, logits)
    m2 = jnp.max(masked, axis=-1, keepdims=True)
    e2 = jnp.min(jnp.where(masked == m2, lane, NUM_EXPERTS), axis=-1,
                 keepdims=True)
    oh2 = (lane == e2)
    oh1f = oh1.astype(jnp.float32)
    oh2f = oh2.astype(jnp.float32)

    a = jnp.exp(m2 - m1)
    wa = 1.0 / (1.0 + a)
    wb = a / (1.0 + a)
    w2_ref[...] = jnp.concatenate([wa, wb], axis=-1)

    z = jnp.exp(logits - m1)
    probs = z / jnp.sum(z, axis=-1, keepdims=True)
    rppe = jnp.mean(probs, axis=0)
    tpe = jnp.mean(oh1f + oh2f, axis=0)
    lbal_ref[0, 0] = NUM_EXPERTS * jnp.sum(tpe * rppe)

    h = oh1f + oh2f
    r_i = lax.broadcasted_iota(jnp.int32, (CSUM_BLK, CSUM_BLK), 0)
    c_i = lax.broadcasted_iota(jnp.int32, (CSUM_BLK, CSUM_BLK), 1)
    tri = (c_i < r_i).astype(jnp.float32)
    carry = jnp.zeros((1, NUM_EXPERTS), jnp.float32)
    excl_chunks = []
    for q in range(N_TOKENS // CSUM_BLK):
        hq = lax.slice_in_dim(h, q * CSUM_BLK, (q + 1) * CSUM_BLK, axis=0)
        excl_chunks.append(
            jnp.dot(tri, hq, preferred_element_type=jnp.float32) + carry)
        carry = carry + jnp.sum(hq, axis=0, keepdims=True)
    excl = jnp.concatenate(excl_chunks, axis=0)
    counts = carry

    cnt_i = counts.astype(jnp.int32)
    cnt_pad = ((cnt_i + (ROW_BLK - 1)) // ROW_BLK) * ROW_BLK
    cnt_pad_f = cnt_pad.astype(jnp.float32)
    r8 = lax.broadcasted_iota(jnp.int32, (NUM_EXPERTS, NUM_EXPERTS), 0)
    c8 = lax.broadcasted_iota(jnp.int32, (NUM_EXPERTS, NUM_EXPERTS), 1)
    strict8 = (r8 < c8).astype(jnp.float32)
    base = jnp.dot(cnt_pad_f, strict8,
                   preferred_element_type=jnp.float32)
    ends = base + cnt_pad_f

    base_b = jnp.broadcast_to(base, excl.shape)
    rank1 = jnp.sum(jnp.where(oh1, excl + base_b, 0.0), axis=-1, keepdims=True)
    rank2 = jnp.sum(jnp.where(oh2, excl + base_b, 0.0), axis=-1, keepdims=True)
    pos_ref[...] = jnp.concatenate([rank1, rank2], axis=-1).astype(jnp.int32)

    blk_start = (lax.broadcasted_iota(jnp.int32, (1, NB), 1)
                 * ROW_BLK).astype(jnp.float32)
    acc = jnp.zeros((1, NB), jnp.int32)
    lane8 = lax.broadcasted_iota(jnp.int32, (1, NUM_EXPERTS), 1)
    for e in range(NUM_EXPERTS):
        end_e = jnp.sum(jnp.where(lane8 == e, ends, 0.0))
        acc = acc + (blk_start >= end_e).astype(jnp.int32)
    gexp_ref[...] = jnp.minimum(acc, NUM_EXPERTS - 1)


def _route(x_flat, Wr):
    return pl.pallas_call(
        _router_kernel,
        out_shape=(
            jax.ShapeDtypeStruct((N_TOKENS, 2), jnp.int32),
            jax.ShapeDtypeStruct((N_TOKENS, 2), jnp.float32),
            jax.ShapeDtypeStruct((1, NB), jnp.int32),
            jax.ShapeDtypeStruct((1, 1), jnp.float32),
        ),
        in_specs=[
            pl.BlockSpec(memory_space=pltpu.VMEM),
            pl.BlockSpec(memory_space=pltpu.VMEM),
        ],
        out_specs=(
            pl.BlockSpec(memory_space=pltpu.VMEM),
            pl.BlockSpec(memory_space=pltpu.VMEM),
            pl.BlockSpec(memory_space=pltpu.VMEM),
            pl.BlockSpec(memory_space=pltpu.SMEM),
        ),
    )(x_flat, Wr)


def _dispatch_body(x_hbm, pos_hbm, w2_hbm, xg_hbm, wg_hbm, idxv, wv, xbuf,
                   sem):
    w = lax.axis_index("s") * 2 + lax.axis_index("c")
    pltpu.sync_copy(pos_hbm.at[0, w], idxv.at[0])
    pltpu.sync_copy(pos_hbm.at[1, w], idxv.at[1])
    pltpu.sync_copy(w2_hbm.at[0, w], wv.at[0])
    pltpu.sync_copy(w2_hbm.at[1, w], wv.at[1])
    for c in range(4):
        pltpu.sync_copy(x_hbm.at[pl.ds(w * TPW + c * 32, 32)], xbuf)
        cp0 = pltpu.async_copy(xbuf, xg_hbm.at[idxv.at[0, c]], sem)
        cp1 = pltpu.async_copy(xbuf, xg_hbm.at[idxv.at[1, c]], sem)
        cp2 = pltpu.async_copy(wv.at[0, c], wg_hbm.at[idxv.at[0, c]], sem)
        cp3 = pltpu.async_copy(wv.at[1, c], wg_hbm.at[idxv.at[1, c]], sem)
        cp0.wait()
        cp1.wait()
        cp2.wait()
        cp3.wait()


def _dispatch(x_flat, posB, w2B):
    mesh = plsc.VectorSubcoreMesh(core_axis_name="c", subcore_axis_name="s",
                                  num_cores=2, num_subcores=16)
    return pl.kernel(
        _dispatch_body,
        out_type=(
            jax.ShapeDtypeStruct((P_ROWS, D_MODEL), jnp.float32),
            jax.ShapeDtypeStruct((P_ROWS,), jnp.float32),
        ),
        mesh=mesh,
        scratch_types=[
            pltpu.VMEM((2, 4, 32), jnp.int32),
            pltpu.VMEM((2, 4, 32), jnp.float32),
            pltpu.VMEM((32, D_MODEL), jnp.float32),
            pltpu.SemaphoreType.DMA,
        ],
    )(x_flat, posB, w2B)


def _gmm_kernel(g_ref, xg_ref, w1_ref, b1_ref, wg_ref, y_ref):
    del g_ref
    y = jnp.dot(xg_ref[...], w1_ref[0], preferred_element_type=jnp.float32)
    y = (y + b1_ref[0]) * wg_ref[...]
    lob = lax.bitcast_convert_type(y[:, :HID_BLK // 2], jnp.int32)
    hib = lax.bitcast_convert_type(y[:, HID_BLK // 2:], jnp.int32)
    lob = jnp.right_shift(lob + 0x8000, 16) & jnp.int32(0xFFFF)
    hib = (hib + 0x8000) & jnp.int32(-65536)
    y_ref[...] = lob | hib


def _grouped_matmul(gexp_flat, xg, W1, b1, wg):
    grid_spec = pltpu.PrefetchScalarGridSpec(
        num_scalar_prefetch=1,
        grid=(HIDDEN // HID_BLK, NB),
        in_specs=[
            pl.BlockSpec((ROW_BLK, D_MODEL), lambda j, i, g: (i, 0)),
            pl.BlockSpec((1, D_MODEL, HID_BLK), lambda j, i, g: (g[i], 0, j)),
            pl.BlockSpec((1, 1, HID_BLK), lambda j, i, g: (g[i], 0, j)),
            pl.BlockSpec((ROW_BLK, 1), lambda j, i, g: (i, 0)),
        ],
        out_specs=pl.BlockSpec((ROW_BLK, HID_BLK // 2),
                               lambda j, i, g: (i, j)),
    )
    return pl.pallas_call(
        _gmm_kernel,
        grid_spec=grid_spec,
        out_shape=jax.ShapeDtypeStruct((P_ROWS, HIDDEN // 2), jnp.int32),
        compiler_params=pltpu.CompilerParams(
            dimension_semantics=("arbitrary", "arbitrary"),
        ),
    )(gexp_flat, xg, W1, b1.reshape(NUM_EXPERTS, 1, HIDDEN),
      wg.reshape(P_ROWS, 1))


NCH = TPW // 4


def _combine_body(y_hbm, pos_hbm, out_hbm, idxv, rA0, rA1, rB0, rB1, o0, o1,
                  sA0, sA1, sB0, sB1, sO0, sO1):
    w = lax.axis_index("s") * 2 + lax.axis_index("c")
    pltpu.sync_copy(pos_hbm.at[0, w], idxv.at[0])
    pltpu.sync_copy(pos_hbm.at[1, w], idxv.at[1])
    rA = (rA0, rA1)
    rB = (rB0, rB1)
    ob = (o0, o1)
    sA = (sA0, sA1)
    sB = (sB0, sB1)
    sO = (sO0, sO1)

    def fire(ch, b):
        pltpu.async_copy(y_hbm.at[idxv.at[0, pl.ds(ch * 4, 4)]], rA[b], sA[b])
        pltpu.async_copy(y_hbm.at[idxv.at[1, pl.ds(ch * 4, 4)]], rB[b], sB[b])

    fire(0, 0)
    fire(1, 1)

    def outer(i, _):
        for b in range(2):
            ch = i * 2 + b
            pltpu.make_async_copy(
                y_hbm.at[idxv.at[0, pl.ds(ch * 4, 4)]], rA[b], sA[b]).wait()
            pltpu.make_async_copy(
                y_hbm.at[idxv.at[1, pl.ds(ch * 4, 4)]], rB[b], sB[b]).wait()

            @pl.when(i > 0)
            def _():
                pltpu.make_async_copy(
                    ob[b], out_hbm.at[pl.ds(w * TPW + (ch - 2) * 4, 4)],
                    sO[b]).wait()

            for t in range(4):

                def elem_body(j, _):
                    for q in range(4):
                        sl = pl.ds(j * 64 + q * 16, 16)
                        slh = pl.ds(HIDDEN // 2 + j * 64 + q * 16, 16)
                        wa = rA[b][t, sl]
                        wb = rB[b][t, sl]
                        alo = lax.bitcast_convert_type(wa << 16, jnp.float32)
                        blo = lax.bitcast_convert_type(wb << 16, jnp.float32)
                        ahi = lax.bitcast_convert_type(
                            wa & jnp.int32(-65536), jnp.float32)
                        bhi = lax.bitcast_convert_type(
                            wb & jnp.int32(-65536), jnp.float32)
                        ob[b][t, sl] = alo + blo
                        ob[b][t, slh] = ahi + bhi
                    return 0

                lax.fori_loop(0, (HIDDEN // 2) // 64, elem_body, 0)

            @pl.when(ch + 2 < NCH)
            def _():
                fire(ch + 2, b)

            pltpu.async_copy(
                ob[b], out_hbm.at[pl.ds(w * TPW + ch * 4, 4)], sO[b])
        return 0

    lax.fori_loop(0, NCH // 2, outer, 0)
    for b in range(2):
        pltpu.make_async_copy(
            ob[b], out_hbm.at[pl.ds(w * TPW + (NCH - 2 + b) * 4, 4)],
            sO[b]).wait()


def _combine(y, posD):
    mesh = plsc.VectorSubcoreMesh(core_axis_name="c", subcore_axis_name="s",
                                  num_cores=2, num_subcores=16)
    return pl.kernel(
        _combine_body,
        out_type=jax.ShapeDtypeStruct((N_TOKENS, HIDDEN), jnp.float32),
        mesh=mesh,
        scratch_types=[
            pltpu.VMEM((2, TPW), jnp.int32),
            pltpu.VMEM((4, HIDDEN // 2), jnp.int32),
            pltpu.VMEM((4, HIDDEN // 2), jnp.int32),
            pltpu.VMEM((4, HIDDEN // 2), jnp.int32),
            pltpu.VMEM((4, HIDDEN // 2), jnp.int32),
            pltpu.VMEM((4, HIDDEN), jnp.float32),
            pltpu.VMEM((4, HIDDEN), jnp.float32),
            pltpu.SemaphoreType.DMA,
            pltpu.SemaphoreType.DMA,
            pltpu.SemaphoreType.DMA,
            pltpu.SemaphoreType.DMA,
            pltpu.SemaphoreType.DMA,
            pltpu.SemaphoreType.DMA,
        ],
    )(y, posD)


def kernel(x, Wr, W1, b1):
    bsz, seq, d = x.shape
    x_flat = x.reshape(N_TOKENS, d)

    pos, w2, gexp, lbal = _route(x_flat, Wr)
    posT = pos.T
    posB = posT.reshape(2, NW, 4, 32)
    posD = posT.reshape(2, NW, TPW)
    w2B = w2.T.reshape(2, NW, 4, 32)

    xg, wg = _dispatch(x_flat, posB, w2B)
    y = _grouped_matmul(gexp.reshape(NB), xg, W1, b1, wg)
    out = _combine(y, posD)
    return out.reshape(bsz, seq, HIDDEN), lbal.reshape(())


# speedup vs baseline: 1.3405x; 1.1685x over previous
import functools

import jax
import jax.numpy as jnp
from jax import lax
from jax.experimental import pallas as pl
from jax.experimental.pallas import tpu as pltpu
from jax.experimental.pallas import tpu_sc as plsc

D_MODEL = 1024
HIDDEN = 4096
NUM_EXPERTS = 8
N_TOKENS = 4096

ROW_BLK = 256
P_ROWS = 2 * N_TOKENS + NUM_EXPERTS * ROW_BLK
NB = P_ROWS // ROW_BLK
HID_BLK = 4096

NW = 32
TPW = N_TOKENS // NW
CSUM_BLK = 512


def _router_kernel(x_ref, wr_ref, pos_ref, w2_ref, gexp_ref, lbal_ref):
    x = x_ref[...]
    wr = wr_ref[...]
    logits = jnp.dot(x, wr, preferred_element_type=jnp.float32)
    lane = lax.broadcasted_iota(jnp.int32, logits.shape, 1)

    m1 = jnp.max(logits, axis=-1, keepdims=True)
    e1 = jnp.min(jnp.where(logits == m1, lane, NUM_EXPERTS), axis=-1,
                 keepdims=True)
    oh1 = (lane == e1)
    masked = jnp.where(oh1, -jnp.inf, logits)
    m2 = jnp.max(masked, axis=-1, keepdims=True)
    e2 = jnp.min(jnp.where(masked == m2, lane, NUM_EXPERTS), axis=-1,
                 keepdims=True)
    oh2 = (lane == e2)
    oh1f = oh1.astype(jnp.float32)
    oh2f = oh2.astype(jnp.float32)

    a = jnp.exp(m2 - m1)
    wa = 1.0 / (1.0 + a)
    wb = a / (1.0 + a)
    w2_ref[...] = jnp.concatenate([wa, wb], axis=-1)

    z = jnp.exp(logits - m1)
    probs = z / jnp.sum(z, axis=-1, keepdims=True)
    rppe = jnp.mean(probs, axis=0)
    tpe = jnp.mean(oh1f + oh2f, axis=0)
    lbal_ref[0, 0] = NUM_EXPERTS * jnp.sum(tpe * rppe)

    h = oh1f + oh2f
    r_i = lax.broadcasted_iota(jnp.int32, (CSUM_BLK, CSUM_BLK), 0)
    c_i = lax.broadcasted_iota(jnp.int32, (CSUM_BLK, CSUM_BLK), 1)
    tri = (c_i < r_i).astype(jnp.float32)
    carry = jnp.zeros((1, NUM_EXPERTS), jnp.float32)
    excl_chunks = []
    for q in range(N_TOKENS // CSUM_BLK):
        hq = lax.slice_in_dim(h, q * CSUM_BLK, (q + 1) * CSUM_BLK, axis=0)
        excl_chunks.append(
            jnp.dot(tri, hq, preferred_element_type=jnp.float32) + carry)
        carry = carry + jnp.sum(hq, axis=0, keepdims=True)
    excl = jnp.concatenate(excl_chunks, axis=0)
    counts = carry

    cnt_i = counts.astype(jnp.int32)
    cnt_pad = ((cnt_i + (ROW_BLK - 1)) // ROW_BLK) * ROW_BLK
    cnt_pad_f = cnt_pad.astype(jnp.float32)
    r8 = lax.broadcasted_iota(jnp.int32, (NUM_EXPERTS, NUM_EXPERTS), 0)
    c8 = lax.broadcasted_iota(jnp.int32, (NUM_EXPERTS, NUM_EXPERTS), 1)
    strict8 = (r8 < c8).astype(jnp.float32)
    base = jnp.dot(cnt_pad_f, strict8,
                   preferred_element_type=jnp.float32)
    ends = base + cnt_pad_f

    base_b = jnp.broadcast_to(base, excl.shape)
    rank1 = jnp.sum(jnp.where(oh1, excl + base_b, 0.0), axis=-1, keepdims=True)
    rank2 = jnp.sum(jnp.where(oh2, excl + base_b, 0.0), axis=-1, keepdims=True)
    pos_ref[...] = jnp.concatenate([rank1, rank2], axis=-1).astype(jnp.int32)

    blk_start = (lax.broadcasted_iota(jnp.int32, (1, NB), 1)
                 * ROW_BLK).astype(jnp.float32)
    acc = jnp.zeros((1, NB), jnp.int32)
    lane8 = lax.broadcasted_iota(jnp.int32, (1, NUM_EXPERTS), 1)
    for e in range(NUM_EXPERTS):
        end_e = jnp.sum(jnp.where(lane8 == e, ends, 0.0))
        acc = acc + (blk_start >= end_e).astype(jnp.int32)
    gexp_ref[...] = jnp.minimum(acc, NUM_EXPERTS - 1)


def _route(x_flat, Wr):
    return pl.pallas_call(
        _router_kernel,
        out_shape=(
            jax.ShapeDtypeStruct((N_TOKENS, 2), jnp.int32),
            jax.ShapeDtypeStruct((N_TOKENS, 2), jnp.float32),
            jax.ShapeDtypeStruct((1, NB), jnp.int32),
            jax.ShapeDtypeStruct((1, 1), jnp.float32),
        ),
        in_specs=[
            pl.BlockSpec(memory_space=pltpu.VMEM),
            pl.BlockSpec(memory_space=pltpu.VMEM),
        ],
        out_specs=(
            pl.BlockSpec(memory_space=pltpu.VMEM),
            pl.BlockSpec(memory_space=pltpu.VMEM),
            pl.BlockSpec(memory_space=pltpu.VMEM),
            pl.BlockSpec(memory_space=pltpu.SMEM),
        ),
    )(x_flat, Wr)


def _dispatch_body(x_hbm, pos_hbm, w2_hbm, xg_hbm, wg_hbm, idxv, wv, xbuf,
                   sem):
    w = lax.axis_index("s") * 2 + lax.axis_index("c")
    pltpu.sync_copy(pos_hbm.at[0, w], idxv.at[0])
    pltpu.sync_copy(pos_hbm.at[1, w], idxv.at[1])
    pltpu.sync_copy(w2_hbm.at[0, w], wv.at[0])
    pltpu.sync_copy(w2_hbm.at[1, w], wv.at[1])
    for c in range(4):
        pltpu.sync_copy(x_hbm.at[pl.ds(w * TPW + c * 32, 32)], xbuf)
        cp0 = pltpu.async_copy(xbuf, xg_hbm.at[idxv.at[0, c]], sem)
        cp1 = pltpu.async_copy(xbuf, xg_hbm.at[idxv.at[1, c]], sem)
        cp2 = pltpu.async_copy(wv.at[0, c], wg_hbm.at[idxv.at[0, c]], sem)
        cp3 = pltpu.async_copy(wv.at[1, c], wg_hbm.at[idxv.at[1, c]], sem)
        cp0.wait()
        cp1.wait()
        cp2.wait()
        cp3.wait()


def _dispatch(x_flat, posB, w2B):
    mesh = plsc.VectorSubcoreMesh(core_axis_name="c", subcore_axis_name="s",
                                  num_cores=2, num_subcores=16)
    return pl.kernel(
        _dispatch_body,
        out_type=(
            jax.ShapeDtypeStruct((P_ROWS, D_MODEL), jnp.float32),
            jax.ShapeDtypeStruct((P_ROWS,), jnp.float32),
        ),
        mesh=mesh,
        scratch_types=[
            pltpu.VMEM((2, 4, 32), jnp.int32),
            pltpu.VMEM((2, 4, 32), jnp.float32),
            pltpu.VMEM((32, D_MODEL), jnp.float32),
            pltpu.SemaphoreType.DMA,
        ],
    )(x_flat, posB, w2B)


def _gmm_kernel(g_ref, xg_ref, w1_ref, b1_ref, wg_ref, y_ref):
    del g_ref
    y = jnp.dot(xg_ref[...], w1_ref[0], preferred_element_type=jnp.float32)
    y = (y + b1_ref[0]) * wg_ref[...]
    lob = lax.bitcast_convert_type(y[:, :HID_BLK // 2], jnp.int32)
    hib = lax.bitcast_convert_type(y[:, HID_BLK // 2:], jnp.int32)
    lob = jnp.right_shift(lob + 0x8000, 16) & jnp.int32(0xFFFF)
    hib = (hib + 0x8000) & jnp.int32(-65536)
    y_ref[...] = lob | hib


def _grouped_matmul(gexp_flat, xg, W1, b1, wg):
    grid_spec = pltpu.PrefetchScalarGridSpec(
        num_scalar_prefetch=1,
        grid=(HIDDEN // HID_BLK, NB),
        in_specs=[
            pl.BlockSpec((ROW_BLK, D_MODEL), lambda j, i, g: (i, 0)),
            pl.BlockSpec((1, D_MODEL, HID_BLK), lambda j, i, g: (g[i], 0, j)),
            pl.BlockSpec((1, 1, HID_BLK), lambda j, i, g: (g[i], 0, j)),
            pl.BlockSpec((ROW_BLK, 1), lambda j, i, g: (i, 0)),
        ],
        out_specs=pl.BlockSpec((ROW_BLK, HID_BLK // 2),
                               lambda j, i, g: (i, j)),
    )
    return pl.pallas_call(
        _gmm_kernel,
        grid_spec=grid_spec,
        out_shape=jax.ShapeDtypeStruct((P_ROWS, HIDDEN // 2), jnp.int32),
        compiler_params=pltpu.CompilerParams(
            dimension_semantics=("arbitrary", "arbitrary"),
        ),
    )(gexp_flat, xg, W1, b1.reshape(NUM_EXPERTS, 1, HIDDEN),
      wg.reshape(P_ROWS, 1))


NCH = TPW // 4


def _combine_body(y_hbm, pos_hbm, out_hbm, idxv, rA0, rA1, rB0, rB1, o0, o1,
                  sA0, sA1, sB0, sB1, sO0, sO1):
    w = lax.axis_index("s") * 2 + lax.axis_index("c")
    pltpu.sync_copy(pos_hbm.at[0, w], idxv.at[0])
    pltpu.sync_copy(pos_hbm.at[1, w], idxv.at[1])
    rA = (rA0, rA1)
    rB = (rB0, rB1)
    ob = (o0, o1)
    sA = (sA0, sA1)
    sB = (sB0, sB1)
    sO = (sO0, sO1)

    def fire(ch, b):
        pltpu.async_copy(y_hbm.at[idxv.at[0, pl.ds(ch * 4, 4)]], rA[b], sA[b])
        pltpu.async_copy(y_hbm.at[idxv.at[1, pl.ds(ch * 4, 4)]], rB[b], sB[b])

    fire(0, 0)
    fire(1, 1)

    def outer(i, _):
        for b in range(2):
            ch = i * 2 + b
            pltpu.make_async_copy(
                y_hbm.at[idxv.at[0, pl.ds(ch * 4, 4)]], rA[b], sA[b]).wait()
            pltpu.make_async_copy(
                y_hbm.at[idxv.at[1, pl.ds(ch * 4, 4)]], rB[b], sB[b]).wait()

            @pl.when(i > 0)
            def _():
                pltpu.make_async_copy(
                    ob[b], out_hbm.at[pl.ds(w * TPW + (ch - 2) * 4, 4)],
                    sO[b]).wait()

            for t in range(4):

                def elem_body(j, _):
                    for q in range(8):
                        sl = pl.ds(j * 128 + q * 16, 16)
                        slh = pl.ds(HIDDEN // 2 + j * 128 + q * 16, 16)
                        wa = rA[b][t, sl]
                        wb = rB[b][t, sl]
                        alo = lax.bitcast_convert_type(wa << 16, jnp.float32)
                        blo = lax.bitcast_convert_type(wb << 16, jnp.float32)
                        ahi = lax.bitcast_convert_type(
                            wa & jnp.int32(-65536), jnp.float32)
                        bhi = lax.bitcast_convert_type(
                            wb & jnp.int32(-65536), jnp.float32)
                        ob[b][t, sl] = alo + blo
                        ob[b][t, slh] = ahi + bhi
                    return 0

                lax.fori_loop(0, (HIDDEN // 2) // 128, elem_body, 0)

            @pl.when(ch + 2 < NCH)
            def _():
                fire(ch + 2, b)

            pltpu.async_copy(
                ob[b], out_hbm.at[pl.ds(w * TPW + ch * 4, 4)], sO[b])
        return 0

    lax.fori_loop(0, NCH // 2, outer, 0)
    for b in range(2):
        pltpu.make_async_copy(
            ob[b], out_hbm.at[pl.ds(w * TPW + (NCH - 2 + b) * 4, 4)],
            sO[b]).wait()


def _combine(y, posD):
    mesh = plsc.VectorSubcoreMesh(core_axis_name="c", subcore_axis_name="s",
                                  num_cores=2, num_subcores=16)
    return pl.kernel(
        _combine_body,
        out_type=jax.ShapeDtypeStruct((N_TOKENS, HIDDEN), jnp.float32),
        mesh=mesh,
        scratch_types=[
            pltpu.VMEM((2, TPW), jnp.int32),
            pltpu.VMEM((4, HIDDEN // 2), jnp.int32),
            pltpu.VMEM((4, HIDDEN // 2), jnp.int32),
            pltpu.VMEM((4, HIDDEN // 2), jnp.int32),
            pltpu.VMEM((4, HIDDEN // 2), jnp.int32),
            pltpu.VMEM((4, HIDDEN), jnp.float32),
            pltpu.VMEM((4, HIDDEN), jnp.float32),
            pltpu.SemaphoreType.DMA,
            pltpu.SemaphoreType.DMA,
            pltpu.SemaphoreType.DMA,
            pltpu.SemaphoreType.DMA,
            pltpu.SemaphoreType.DMA,
            pltpu.SemaphoreType.DMA,
        ],
    )(y, posD)


def kernel(x, Wr, W1, b1):
    bsz, seq, d = x.shape
    x_flat = x.reshape(N_TOKENS, d)

    pos, w2, gexp, lbal = _route(x_flat, Wr)
    posT = pos.T
    posB = posT.reshape(2, NW, 4, 32)
    posD = posT.reshape(2, NW, TPW)
    w2B = w2.T.reshape(2, NW, 4, 32)

    xg, wg = _dispatch(x_flat, posB, w2B)
    y = _grouped_matmul(gexp.reshape(NB), xg, W1, b1, wg)
    out = _combine(y, posD)
    return out.reshape(bsz, seq, HIDDEN), lbal.reshape(())


# speedup vs baseline: 1.3486x; 1.0060x over previous
import functools

import jax
import jax.numpy as jnp
from jax import lax
from jax.experimental import pallas as pl
from jax.experimental.pallas import tpu as pltpu
from jax.experimental.pallas import tpu_sc as plsc

D_MODEL = 1024
HIDDEN = 4096
NUM_EXPERTS = 8
N_TOKENS = 4096

ROW_BLK = 256
P_ROWS = 2 * N_TOKENS + NUM_EXPERTS * ROW_BLK
NB = P_ROWS // ROW_BLK
HID_BLK = 4096

NW = 32
TPW = N_TOKENS // NW
CSUM_BLK = 512


def _router_kernel(x_ref, wr_ref, pos_ref, w2_ref, gexp_ref, lbal_ref,
                   xi_ref):
    x = x_ref[...]
    wr = wr_ref[...]
    lob = lax.bitcast_convert_type(x[:, :D_MODEL // 2], jnp.int32)
    hib = lax.bitcast_convert_type(x[:, D_MODEL // 2:], jnp.int32)
    lob = jnp.right_shift(lob + 0x8000, 16) & jnp.int32(0xFFFF)
    hib = (hib + 0x8000) & jnp.int32(-65536)
    xi_ref[...] = lob | hib
    logits = jnp.dot(x, wr, preferred_element_type=jnp.float32)
    lane = lax.broadcasted_iota(jnp.int32, logits.shape, 1)

    m1 = jnp.max(logits, axis=-1, keepdims=True)
    e1 = jnp.min(jnp.where(logits == m1, lane, NUM_EXPERTS), axis=-1,
                 keepdims=True)
    oh1 = (lane == e1)
    masked = jnp.where(oh1, -jnp.inf, logits)
    m2 = jnp.max(masked, axis=-1, keepdims=True)
    e2 = jnp.min(jnp.where(masked == m2, lane, NUM_EXPERTS), axis=-1,
                 keepdims=True)
    oh2 = (lane == e2)
    oh1f = oh1.astype(jnp.float32)
    oh2f = oh2.astype(jnp.float32)

    a = jnp.exp(m2 - m1)
    wa = 1.0 / (1.0 + a)
    wb = a / (1.0 + a)
    w2_ref[...] = jnp.concatenate([wa, wb], axis=-1)

    z = jnp.exp(logits - m1)
    probs = z / jnp.sum(z, axis=-1, keepdims=True)
    rppe = jnp.mean(probs, axis=0)
    tpe = jnp.mean(oh1f + oh2f, axis=0)
    lbal_ref[0, 0] = NUM_EXPERTS * jnp.sum(tpe * rppe)

    h = oh1f + oh2f
    r_i = lax.broadcasted_iota(jnp.int32, (CSUM_BLK, CSUM_BLK), 0)
    c_i = lax.broadcasted_iota(jnp.int32, (CSUM_BLK, CSUM_BLK), 1)
    tri = (c_i < r_i).astype(jnp.float32)
    carry = jnp.zeros((1, NUM_EXPERTS), jnp.float32)
    excl_chunks = []
    for q in range(N_TOKENS // CSUM_BLK):
        hq = lax.slice_in_dim(h, q * CSUM_BLK, (q + 1) * CSUM_BLK, axis=0)
        excl_chunks.append(
            jnp.dot(tri, hq, preferred_element_type=jnp.float32) + carry)
        carry = carry + jnp.sum(hq, axis=0, keepdims=True)
    excl = jnp.concatenate(excl_chunks, axis=0)
    counts = carry

    cnt_i = counts.astype(jnp.int32)
    cnt_pad = ((cnt_i + (ROW_BLK - 1)) // ROW_BLK) * ROW_BLK
    cnt_pad_f = cnt_pad.astype(jnp.float32)
    r8 = lax.broadcasted_iota(jnp.int32, (NUM_EXPERTS, NUM_EXPERTS), 0)
    c8 = lax.broadcasted_iota(jnp.int32, (NUM_EXPERTS, NUM_EXPERTS), 1)
    strict8 = (r8 < c8).astype(jnp.float32)
    base = jnp.dot(cnt_pad_f, strict8,
                   preferred_element_type=jnp.float32)
    ends = base + cnt_pad_f

    base_b = jnp.broadcast_to(base, excl.shape)
    rank1 = jnp.sum(jnp.where(oh1, excl + base_b, 0.0), axis=-1, keepdims=True)
    rank2 = jnp.sum(jnp.where(oh2, excl + base_b, 0.0), axis=-1, keepdims=True)
    pos_ref[...] = jnp.concatenate([rank1, rank2], axis=-1).astype(jnp.int32)

    blk_start = (lax.broadcasted_iota(jnp.int32, (1, NB), 1)
                 * ROW_BLK).astype(jnp.float32)
    acc = jnp.zeros((1, NB), jnp.int32)
    lane8 = lax.broadcasted_iota(jnp.int32, (1, NUM_EXPERTS), 1)
    for e in range(NUM_EXPERTS):
        end_e = jnp.sum(jnp.where(lane8 == e, ends, 0.0))
        acc = acc + (blk_start >= end_e).astype(jnp.int32)
    gexp_ref[...] = jnp.minimum(acc, NUM_EXPERTS - 1)


def _route(x_flat, Wr):
    return pl.pallas_call(
        _router_kernel,
        out_shape=(
            jax.ShapeDtypeStruct((N_TOKENS, 2), jnp.int32),
            jax.ShapeDtypeStruct((N_TOKENS, 2), jnp.float32),
            jax.ShapeDtypeStruct((1, NB), jnp.int32),
            jax.ShapeDtypeStruct((1, 1), jnp.float32),
            jax.ShapeDtypeStruct((N_TOKENS, D_MODEL // 2), jnp.int32),
        ),
        in_specs=[
            pl.BlockSpec(memory_space=pltpu.VMEM),
            pl.BlockSpec(memory_space=pltpu.VMEM),
        ],
        out_specs=(
            pl.BlockSpec(memory_space=pltpu.VMEM),
            pl.BlockSpec(memory_space=pltpu.VMEM),
            pl.BlockSpec(memory_space=pltpu.VMEM),
            pl.BlockSpec(memory_space=pltpu.SMEM),
            pl.BlockSpec(memory_space=pltpu.VMEM),
        ),
    )(x_flat, Wr)


def _dispatch_body(xi_hbm, pos_hbm, w2_hbm, xg_hbm, wg_hbm, idxv, wv, xbuf,
                   sem):
    w = lax.axis_index("s") * 2 + lax.axis_index("c")
    pltpu.sync_copy(pos_hbm.at[0, w], idxv.at[0])
    pltpu.sync_copy(pos_hbm.at[1, w], idxv.at[1])
    pltpu.sync_copy(w2_hbm.at[0, w], wv.at[0])
    pltpu.sync_copy(w2_hbm.at[1, w], wv.at[1])
    for c in range(4):
        pltpu.sync_copy(xi_hbm.at[pl.ds(w * TPW + c * 32, 32)], xbuf)
        cp0 = pltpu.async_copy(xbuf, xg_hbm.at[idxv.at[0, c]], sem)
        cp1 = pltpu.async_copy(xbuf, xg_hbm.at[idxv.at[1, c]], sem)
        cp2 = pltpu.async_copy(wv.at[0, c], wg_hbm.at[idxv.at[0, c]], sem)
        cp3 = pltpu.async_copy(wv.at[1, c], wg_hbm.at[idxv.at[1, c]], sem)
        cp0.wait()
        cp1.wait()
        cp2.wait()
        cp3.wait()


def _dispatch(xi, posB, w2B):
    mesh = plsc.VectorSubcoreMesh(core_axis_name="c", subcore_axis_name="s",
                                  num_cores=2, num_subcores=16)
    return pl.kernel(
        _dispatch_body,
        out_type=(
            jax.ShapeDtypeStruct((P_ROWS, D_MODEL // 2), jnp.int32),
            jax.ShapeDtypeStruct((P_ROWS,), jnp.float32),
        ),
        mesh=mesh,
        scratch_types=[
            pltpu.VMEM((2, 4, 32), jnp.int32),
            pltpu.VMEM((2, 4, 32), jnp.float32),
            pltpu.VMEM((32, D_MODEL // 2), jnp.int32),
            pltpu.SemaphoreType.DMA,
        ],
    )(xi, posB, w2B)


def _gmm_kernel(g_ref, xg_ref, w1_ref, b1_ref, wg_ref, y_ref):
    del g_ref
    xi = xg_ref[...]
    xlo = lax.bitcast_convert_type(xi << 16, jnp.float32)
    xhi = lax.bitcast_convert_type(xi & jnp.int32(-65536), jnp.float32)
    w1 = w1_ref[0]
    y = (jnp.dot(xlo, w1[:D_MODEL // 2], preferred_element_type=jnp.float32)
         + jnp.dot(xhi, w1[D_MODEL // 2:], preferred_element_type=jnp.float32))
    y = (y + b1_ref[0]) * wg_ref[...]
    lob = lax.bitcast_convert_type(y[:, :HID_BLK // 2], jnp.int32)
    hib = lax.bitcast_convert_type(y[:, HID_BLK // 2:], jnp.int32)
    lob = jnp.right_shift(lob + 0x8000, 16) & jnp.int32(0xFFFF)
    hib = (hib + 0x8000) & jnp.int32(-65536)
    y_ref[...] = lob | hib


def _grouped_matmul(gexp_flat, xg, W1, b1, wg):
    grid_spec = pltpu.PrefetchScalarGridSpec(
        num_scalar_prefetch=1,
        grid=(HIDDEN // HID_BLK, NB),
        in_specs=[
            pl.BlockSpec((ROW_BLK, D_MODEL // 2), lambda j, i, g: (i, 0)),
            pl.BlockSpec((1, D_MODEL, HID_BLK), lambda j, i, g: (g[i], 0, j)),
            pl.BlockSpec((1, 1, HID_BLK), lambda j, i, g: (g[i], 0, j)),
            pl.BlockSpec((ROW_BLK, 1), lambda j, i, g: (i, 0)),
        ],
        out_specs=pl.BlockSpec((ROW_BLK, HID_BLK // 2),
                               lambda j, i, g: (i, j)),
    )
    return pl.pallas_call(
        _gmm_kernel,
        grid_spec=grid_spec,
        out_shape=jax.ShapeDtypeStruct((P_ROWS, HIDDEN // 2), jnp.int32),
        compiler_params=pltpu.CompilerParams(
            dimension_semantics=("arbitrary", "arbitrary"),
        ),
    )(gexp_flat, xg, W1, b1.reshape(NUM_EXPERTS, 1, HIDDEN),
      wg.reshape(P_ROWS, 1))


NCH = TPW // 4


def _combine_body(y_hbm, pos_hbm, out_hbm, idxv, rA0, rA1, rB0, rB1, o0, o1,
                  sA0, sA1, sB0, sB1, sO0, sO1):
    w = lax.axis_index("s") * 2 + lax.axis_index("c")
    pltpu.sync_copy(pos_hbm.at[0, w], idxv.at[0])
    pltpu.sync_copy(pos_hbm.at[1, w], idxv.at[1])
    rA = (rA0, rA1)
    rB = (rB0, rB1)
    ob = (o0, o1)
    sA = (sA0, sA1)
    sB = (sB0, sB1)
    sO = (sO0, sO1)

    def fire(ch, b):
        pltpu.async_copy(y_hbm.at[idxv.at[0, pl.ds(ch * 4, 4)]], rA[b], sA[b])
        pltpu.async_copy(y_hbm.at[idxv.at[1, pl.ds(ch * 4, 4)]], rB[b], sB[b])

    fire(0, 0)
    fire(1, 1)

    def outer(i, _):
        for b in range(2):
            ch = i * 2 + b
            pltpu.make_async_copy(
                y_hbm.at[idxv.at[0, pl.ds(ch * 4, 4)]], rA[b], sA[b]).wait()
            pltpu.make_async_copy(
                y_hbm.at[idxv.at[1, pl.ds(ch * 4, 4)]], rB[b], sB[b]).wait()

            @pl.when(i > 0)
            def _():
                pltpu.make_async_copy(
                    ob[b], out_hbm.at[pl.ds(w * TPW + (ch - 2) * 4, 4)],
                    sO[b]).wait()

            for t in range(4):

                def elem_body(j, _):
                    for q in range(8):
                        sl = pl.ds(j * 128 + q * 16, 16)
                        slh = pl.ds(HIDDEN // 2 + j * 128 + q * 16, 16)
                        wa = rA[b][t, sl]
                        wb = rB[b][t, sl]
                        alo = lax.bitcast_convert_type(wa << 16, jnp.float32)
                        blo = lax.bitcast_convert_type(wb << 16, jnp.float32)
                        ahi = lax.bitcast_convert_type(
                            wa & jnp.int32(-65536), jnp.float32)
                        bhi = lax.bitcast_convert_type(
                            wb & jnp.int32(-65536), jnp.float32)
                        ob[b][t, sl] = alo + blo
                        ob[b][t, slh] = ahi + bhi
                    return 0

                lax.fori_loop(0, (HIDDEN // 2) // 128, elem_body, 0)

            @pl.when(ch + 2 < NCH)
            def _():
                fire(ch + 2, b)

            pltpu.async_copy(
                ob[b], out_hbm.at[pl.ds(w * TPW + ch * 4, 4)], sO[b])
        return 0

    lax.fori_loop(0, NCH // 2, outer, 0)
    for b in range(2):
        pltpu.make_async_copy(
            ob[b], out_hbm.at[pl.ds(w * TPW + (NCH - 2 + b) * 4, 4)],
            sO[b]).wait()


def _combine(y, posD):
    mesh = plsc.VectorSubcoreMesh(core_axis_name="c", subcore_axis_name="s",
                                  num_cores=2, num_subcores=16)
    return pl.kernel(
        _combine_body,
        out_type=jax.ShapeDtypeStruct((N_TOKENS, HIDDEN), jnp.float32),
        mesh=mesh,
        scratch_types=[
            pltpu.VMEM((2, TPW), jnp.int32),
            pltpu.VMEM((4, HIDDEN // 2), jnp.int32),
            pltpu.VMEM((4, HIDDEN // 2), jnp.int32),
            pltpu.VMEM((4, HIDDEN // 2), jnp.int32),
            pltpu.VMEM((4, HIDDEN // 2), jnp.int32),
            pltpu.VMEM((4, HIDDEN), jnp.float32),
            pltpu.VMEM((4, HIDDEN), jnp.float32),
            pltpu.SemaphoreType.DMA,
            pltpu.SemaphoreType.DMA,
            pltpu.SemaphoreType.DMA,
            pltpu.SemaphoreType.DMA,
            pltpu.SemaphoreType.DMA,
            pltpu.SemaphoreType.DMA,
        ],
    )(y, posD)


def kernel(x, Wr, W1, b1):
    bsz, seq, d = x.shape
    x_flat = x.reshape(N_TOKENS, d)

    pos, w2, gexp, lbal, xi = _route(x_flat, Wr)
    posT = pos.T
    posB = posT.reshape(2, NW, 4, 32)
    posD = posT.reshape(2, NW, TPW)
    w2B = w2.T.reshape(2, NW, 4, 32)

    xg, wg = _dispatch(xi, posB, w2B)
    y = _grouped_matmul(gexp.reshape(NB), xg, W1, b1, wg)
    out = _combine(y, posD)
    return out.reshape(bsz, seq, HIDDEN), lbal.reshape(())


# speedup vs baseline: 1.3508x; 1.0016x over previous
import functools

import jax
import jax.numpy as jnp
from jax import lax
from jax.experimental import pallas as pl
from jax.experimental.pallas import tpu as pltpu
from jax.experimental.pallas import tpu_sc as plsc

D_MODEL = 1024
HIDDEN = 4096
NUM_EXPERTS = 8
N_TOKENS = 4096

ROW_BLK = 256
P_ROWS = 2 * N_TOKENS + NUM_EXPERTS * ROW_BLK
NB = P_ROWS // ROW_BLK
HID_BLK = 4096

NW = 32
TPW = N_TOKENS // NW
CSUM_BLK = 512


def _router_kernel(x_ref, wr_ref, pos_ref, w2_ref, gexp_ref, lbal_ref,
                   xi_ref):
    x = x_ref[...]
    wr = wr_ref[...]
    lob = lax.bitcast_convert_type(x[:, :D_MODEL // 2], jnp.int32)
    hib = lax.bitcast_convert_type(x[:, D_MODEL // 2:], jnp.int32)
    lob = jnp.right_shift(lob + 0x8000, 16) & jnp.int32(0xFFFF)
    hib = (hib + 0x8000) & jnp.int32(-65536)
    xi_ref[...] = lob | hib
    logits = jnp.dot(x, wr, preferred_element_type=jnp.float32)
    lane = lax.broadcasted_iota(jnp.int32, logits.shape, 1)

    m1 = jnp.max(logits, axis=-1, keepdims=True)
    e1 = jnp.min(jnp.where(logits == m1, lane, NUM_EXPERTS), axis=-1,
                 keepdims=True)
    oh1 = (lane == e1)
    masked = jnp.where(oh1, -jnp.inf, logits)
    m2 = jnp.max(masked, axis=-1, keepdims=True)
    e2 = jnp.min(jnp.where(masked == m2, lane, NUM_EXPERTS), axis=-1,
                 keepdims=True)
    oh2 = (lane == e2)
    oh1f = oh1.astype(jnp.float32)
    oh2f = oh2.astype(jnp.float32)

    a = jnp.exp(m2 - m1)
    wa = 1.0 / (1.0 + a)
    wb = a / (1.0 + a)
    w2_ref[...] = jnp.concatenate([wa, wb], axis=-1)

    z = jnp.exp(logits - m1)
    probs = z / jnp.sum(z, axis=-1, keepdims=True)
    rppe = jnp.mean(probs, axis=0)
    tpe = jnp.mean(oh1f + oh2f, axis=0)
    lbal_ref[0, 0] = NUM_EXPERTS * jnp.sum(tpe * rppe)

    h = oh1f + oh2f
    r_i = lax.broadcasted_iota(jnp.int32, (CSUM_BLK, CSUM_BLK), 0)
    c_i = lax.broadcasted_iota(jnp.int32, (CSUM_BLK, CSUM_BLK), 1)
    tri = (c_i < r_i).astype(jnp.float32)
    carry = jnp.zeros((1, NUM_EXPERTS), jnp.float32)
    excl_chunks = []
    for q in range(N_TOKENS // CSUM_BLK):
        hq = lax.slice_in_dim(h, q * CSUM_BLK, (q + 1) * CSUM_BLK, axis=0)
        excl_chunks.append(
            jnp.dot(tri, hq, preferred_element_type=jnp.float32) + carry)
        carry = carry + jnp.sum(hq, axis=0, keepdims=True)
    excl = jnp.concatenate(excl_chunks, axis=0)
    counts = carry

    cnt_i = counts.astype(jnp.int32)
    cnt_pad = ((cnt_i + (ROW_BLK - 1)) // ROW_BLK) * ROW_BLK
    cnt_pad_f = cnt_pad.astype(jnp.float32)
    r8 = lax.broadcasted_iota(jnp.int32, (NUM_EXPERTS, NUM_EXPERTS), 0)
    c8 = lax.broadcasted_iota(jnp.int32, (NUM_EXPERTS, NUM_EXPERTS), 1)
    strict8 = (r8 < c8).astype(jnp.float32)
    base = jnp.dot(cnt_pad_f, strict8,
                   preferred_element_type=jnp.float32)
    ends = base + cnt_pad_f

    base_b = jnp.broadcast_to(base, excl.shape)
    rank1 = jnp.sum(jnp.where(oh1, excl + base_b, 0.0), axis=-1, keepdims=True)
    rank2 = jnp.sum(jnp.where(oh2, excl + base_b, 0.0), axis=-1, keepdims=True)
    pos_ref[...] = jnp.concatenate([rank1, rank2], axis=-1).astype(jnp.int32)

    blk_start = (lax.broadcasted_iota(jnp.int32, (1, NB), 1)
                 * ROW_BLK).astype(jnp.float32)
    acc = jnp.zeros((1, NB), jnp.int32)
    lane8 = lax.broadcasted_iota(jnp.int32, (1, NUM_EXPERTS), 1)
    for e in range(NUM_EXPERTS):
        end_e = jnp.sum(jnp.where(lane8 == e, ends, 0.0))
        acc = acc + (blk_start >= end_e).astype(jnp.int32)
    gexp_ref[...] = jnp.minimum(acc, NUM_EXPERTS - 1)


def _route(x_flat, Wr):
    return pl.pallas_call(
        _router_kernel,
        out_shape=(
            jax.ShapeDtypeStruct((N_TOKENS, 2), jnp.int32),
            jax.ShapeDtypeStruct((N_TOKENS, 2), jnp.float32),
            jax.ShapeDtypeStruct((1, NB), jnp.int32),
            jax.ShapeDtypeStruct((1, 1), jnp.float32),
            jax.ShapeDtypeStruct((N_TOKENS, D_MODEL // 2), jnp.int32),
        ),
        in_specs=[
            pl.BlockSpec(memory_space=pltpu.VMEM),
            pl.BlockSpec(memory_space=pltpu.VMEM),
        ],
        out_specs=(
            pl.BlockSpec(memory_space=pltpu.VMEM),
            pl.BlockSpec(memory_space=pltpu.VMEM),
            pl.BlockSpec(memory_space=pltpu.VMEM),
            pl.BlockSpec(memory_space=pltpu.SMEM),
            pl.BlockSpec(memory_space=pltpu.VMEM),
        ),
    )(x_flat, Wr)


def _dispatch_body(xi_hbm, pos_hbm, w2_hbm, xg_hbm, wg_hbm, idxv, wv, x0, x1,
                   x2, x3, semR, semS):
    w = lax.axis_index("s") * 2 + lax.axis_index("c")
    pltpu.sync_copy(pos_hbm.at[0, w], idxv.at[0])
    pltpu.sync_copy(pos_hbm.at[1, w], idxv.at[1])
    pltpu.sync_copy(w2_hbm.at[0, w], wv.at[0])
    pltpu.sync_copy(w2_hbm.at[1, w], wv.at[1])
    xb = (x0, x1, x2, x3)
    for c in range(4):
        pltpu.async_copy(xi_hbm.at[pl.ds(w * TPW + c * 32, 32)], xb[c], semR)
    for c in range(4):
        pltpu.make_async_copy(
            xi_hbm.at[pl.ds(w * TPW + c * 32, 32)], xb[c], semR).wait()
        pltpu.async_copy(xb[c], xg_hbm.at[idxv.at[0, c]], semS)
        pltpu.async_copy(xb[c], xg_hbm.at[idxv.at[1, c]], semS)
        pltpu.async_copy(wv.at[0, c], wg_hbm.at[idxv.at[0, c]], semS)
        pltpu.async_copy(wv.at[1, c], wg_hbm.at[idxv.at[1, c]], semS)
    for c in range(4):
        pltpu.make_async_copy(xb[c], xg_hbm.at[idxv.at[0, c]], semS).wait()
        pltpu.make_async_copy(xb[c], xg_hbm.at[idxv.at[1, c]], semS).wait()
        pltpu.make_async_copy(wv.at[0, c], wg_hbm.at[idxv.at[0, c]],
                              semS).wait()
        pltpu.make_async_copy(wv.at[1, c], wg_hbm.at[idxv.at[1, c]],
                              semS).wait()


def _dispatch(xi, posB, w2B):
    mesh = plsc.VectorSubcoreMesh(core_axis_name="c", subcore_axis_name="s",
                                  num_cores=2, num_subcores=16)
    return pl.kernel(
        _dispatch_body,
        out_type=(
            jax.ShapeDtypeStruct((P_ROWS, D_MODEL // 2), jnp.int32),
            jax.ShapeDtypeStruct((P_ROWS,), jnp.float32),
        ),
        mesh=mesh,
        scratch_types=[
            pltpu.VMEM((2, 4, 32), jnp.int32),
            pltpu.VMEM((2, 4, 32), jnp.float32),
            pltpu.VMEM((32, D_MODEL // 2), jnp.int32),
            pltpu.VMEM((32, D_MODEL // 2), jnp.int32),
            pltpu.VMEM((32, D_MODEL // 2), jnp.int32),
            pltpu.VMEM((32, D_MODEL // 2), jnp.int32),
            pltpu.SemaphoreType.DMA,
            pltpu.SemaphoreType.DMA,
        ],
    )(xi, posB, w2B)


def _gmm_kernel(g_ref, xg_ref, w1_ref, b1_ref, wg_ref, y_ref):
    del g_ref
    xi = xg_ref[...]
    xlo = lax.bitcast_convert_type(xi << 16, jnp.float32)
    xhi = lax.bitcast_convert_type(xi & jnp.int32(-65536), jnp.float32)
    w1 = w1_ref[0]
    y = (jnp.dot(xlo, w1[:D_MODEL // 2], preferred_element_type=jnp.float32)
         + jnp.dot(xhi, w1[D_MODEL // 2:], preferred_element_type=jnp.float32))
    y = (y + b1_ref[0]) * wg_ref[...]
    lob = lax.bitcast_convert_type(y[:, :HID_BLK // 2], jnp.int32)
    hib = lax.bitcast_convert_type(y[:, HID_BLK // 2:], jnp.int32)
    lob = jnp.right_shift(lob + 0x8000, 16) & jnp.int32(0xFFFF)
    hib = (hib + 0x8000) & jnp.int32(-65536)
    y_ref[...] = lob | hib


def _grouped_matmul(gexp_flat, xg, W1, b1, wg):
    grid_spec = pltpu.PrefetchScalarGridSpec(
        num_scalar_prefetch=1,
        grid=(HIDDEN // HID_BLK, NB),
        in_specs=[
            pl.BlockSpec((ROW_BLK, D_MODEL // 2), lambda j, i, g: (i, 0)),
            pl.BlockSpec((1, D_MODEL, HID_BLK), lambda j, i, g: (g[i], 0, j)),
            pl.BlockSpec((1, 1, HID_BLK), lambda j, i, g: (g[i], 0, j)),
            pl.BlockSpec((ROW_BLK, 1), lambda j, i, g: (i, 0)),
        ],
        out_specs=pl.BlockSpec((ROW_BLK, HID_BLK // 2),
                               lambda j, i, g: (i, j)),
    )
    return pl.pallas_call(
        _gmm_kernel,
        grid_spec=grid_spec,
        out_shape=jax.ShapeDtypeStruct((P_ROWS, HIDDEN // 2), jnp.int32),
        compiler_params=pltpu.CompilerParams(
            dimension_semantics=("arbitrary", "arbitrary"),
        ),
    )(gexp_flat, xg, W1, b1.reshape(NUM_EXPERTS, 1, HIDDEN),
      wg.reshape(P_ROWS, 1))


NCH = TPW // 4


def _combine_body(y_hbm, pos_hbm, out_hbm, idxv, rA0, rA1, rB0, rB1, o0, o1,
                  sA0, sA1, sB0, sB1, sO0, sO1):
    w = lax.axis_index("s") * 2 + lax.axis_index("c")
    pltpu.sync_copy(pos_hbm.at[0, w], idxv.at[0])
    pltpu.sync_copy(pos_hbm.at[1, w], idxv.at[1])
    rA = (rA0, rA1)
    rB = (rB0, rB1)
    ob = (o0, o1)
    sA = (sA0, sA1)
    sB = (sB0, sB1)
    sO = (sO0, sO1)

    def fire(ch, b):
        pltpu.async_copy(y_hbm.at[idxv.at[0, pl.ds(ch * 4, 4)]], rA[b], sA[b])
        pltpu.async_copy(y_hbm.at[idxv.at[1, pl.ds(ch * 4, 4)]], rB[b], sB[b])

    fire(0, 0)
    fire(1, 1)

    def outer(i, _):
        for b in range(2):
            ch = i * 2 + b
            pltpu.make_async_copy(
                y_hbm.at[idxv.at[0, pl.ds(ch * 4, 4)]], rA[b], sA[b]).wait()
            pltpu.make_async_copy(
                y_hbm.at[idxv.at[1, pl.ds(ch * 4, 4)]], rB[b], sB[b]).wait()

            @pl.when(i > 0)
            def _():
                pltpu.make_async_copy(
                    ob[b], out_hbm.at[pl.ds(w * TPW + (ch - 2) * 4, 4)],
                    sO[b]).wait()

            for t in range(4):

                def elem_body(j, _):
                    for q in range(8):
                        sl = pl.ds(j * 128 + q * 16, 16)
                        slh = pl.ds(HIDDEN // 2 + j * 128 + q * 16, 16)
                        wa = rA[b][t, sl]
                        wb = rB[b][t, sl]
                        alo = lax.bitcast_convert_type(wa << 16, jnp.float32)
                        blo = lax.bitcast_convert_type(wb << 16, jnp.float32)
                        ahi = lax.bitcast_convert_type(
                            wa & jnp.int32(-65536), jnp.float32)
                        bhi = lax.bitcast_convert_type(
                            wb & jnp.int32(-65536), jnp.float32)
                        ob[b][t, sl] = alo + blo
                        ob[b][t, slh] = ahi + bhi
                    return 0

                lax.fori_loop(0, (HIDDEN // 2) // 128, elem_body, 0)

            @pl.when(ch + 2 < NCH)
            def _():
                fire(ch + 2, b)

            pltpu.async_copy(
                ob[b], out_hbm.at[pl.ds(w * TPW + ch * 4, 4)], sO[b])
        return 0

    lax.fori_loop(0, NCH // 2, outer, 0)
    for b in range(2):
        pltpu.make_async_copy(
            ob[b], out_hbm.at[pl.ds(w * TPW + (NCH - 2 + b) * 4, 4)],
            sO[b]).wait()


def _combine(y, posD):
    mesh = plsc.VectorSubcoreMesh(core_axis_name="c", subcore_axis_name="s",
                                  num_cores=2, num_subcores=16)
    return pl.kernel(
        _combine_body,
        out_type=jax.ShapeDtypeStruct((N_TOKENS, HIDDEN), jnp.float32),
        mesh=mesh,
        scratch_types=[
            pltpu.VMEM((2, TPW), jnp.int32),
            pltpu.VMEM((4, HIDDEN // 2), jnp.int32),
            pltpu.VMEM((4, HIDDEN // 2), jnp.int32),
            pltpu.VMEM((4, HIDDEN // 2), jnp.int32),
            pltpu.VMEM((4, HIDDEN // 2), jnp.int32),
            pltpu.VMEM((4, HIDDEN), jnp.float32),
            pltpu.VMEM((4, HIDDEN), jnp.float32),
            pltpu.SemaphoreType.DMA,
            pltpu.SemaphoreType.DMA,
            pltpu.SemaphoreType.DMA,
            pltpu.SemaphoreType.DMA,
            pltpu.SemaphoreType.DMA,
            pltpu.SemaphoreType.DMA,
        ],
    )(y, posD)


def kernel(x, Wr, W1, b1):
    bsz, seq, d = x.shape
    x_flat = x.reshape(N_TOKENS, d)

    pos, w2, gexp, lbal, xi = _route(x_flat, Wr)
    posT = pos.T
    posB = posT.reshape(2, NW, 4, 32)
    posD = posT.reshape(2, NW, TPW)
    w2B = w2.T.reshape(2, NW, 4, 32)

    xg, wg = _dispatch(xi, posB, w2B)
    y = _grouped_matmul(gexp.reshape(NB), xg, W1, b1, wg)
    out = _combine(y, posD)
    return out.reshape(bsz, seq, HIDDEN), lbal.reshape(())
